# Initial kernel scaffold; baseline (speedup 1.0000x reference)
import jax
import jax.numpy as jnp
from jax.experimental import pallas as pl


def kernel(numeric, categorical, W_num, b_num, fo_tables, so_tables, W0, b0, W1, b1, W2, b2, Wout, bout):
    raise NotImplementedError("write your pallas kernel here")


# speedup vs baseline: 17.4113x; 17.4113x over previous
import functools

import jax
import jax.numpy as jnp
from jax import lax
from jax.experimental import pallas as pl
from jax.experimental.pallas import tpu as pltpu
from jax.experimental.pallas import tpu_sc as plsc

_INFO = plsc.get_sparse_core_info()
_NC = _INFO.num_cores
_NS = _INFO.num_subcores
_NW = _NC * _NS


def _sc_gather_fn(B, F, V, D, fo_pad_len):
    rows_w = B // _NW
    idx_w = rows_w * F
    assert idx_w % 128 == 0
    n_chunks = idx_w // 128
    q_per_w = rows_w // 128
    n_fo = rows_w // 16

    mesh = plsc.VectorSubcoreMesh(core_axis_name="c", subcore_axis_name="s")

    def body(idx_rm_hbm, idx_fm_hbm, fo_hbm, so_hbm, flat_out, fo_out,
             idx_rm_v, idx_fm_v, rows_v, val_v, acc_v, sem):
        wid = lax.axis_index("s") * _NC + lax.axis_index("c")
        pltpu.sync_copy(idx_rm_hbm.at[wid], idx_rm_v)
        pltpu.sync_copy(idx_fm_hbm.at[wid], idx_fm_v)

        base_bf = wid * idx_w

        def gchunk(c, carry):
            pltpu.async_copy(so_hbm.at[idx_rm_v.at[c]], rows_v, sem).wait()
            pltpu.sync_copy(rows_v, flat_out.at[pl.ds(base_bf + c * 128, 128)])
            return carry
        lax.fori_loop(0, n_chunks, gchunk, 0)

        def zinit(j, carry):
            acc_v[pl.ds(j * 16, 16)] = jnp.zeros((16,), jnp.float32)
            return carry
        lax.fori_loop(0, n_fo, zinit, 0)

        def fochunk(t, carry):
            f = t // q_per_w
            q = t % q_per_w
            pltpu.async_copy(fo_hbm.at[idx_fm_v.at[f, q]], val_v, sem).wait()
            for k in range(8):
                sl = pl.ds(q * 128 + k * 16, 16)
                acc_v[sl] = acc_v[sl] + val_v[pl.ds(k * 16, 16)]
            return carry
        lax.fori_loop(0, F * q_per_w, fochunk, 0)
        pltpu.sync_copy(acc_v, fo_out.at[pl.ds(wid * rows_w, rows_w)])

    return pl.kernel(
        body,
        mesh=mesh,
        out_type=(
            jax.ShapeDtypeStruct((B * F, D), jnp.float32),
            jax.ShapeDtypeStruct((B,), jnp.float32),
        ),
        scratch_types=[
            pltpu.VMEM((n_chunks, 128), jnp.int32),
            pltpu.VMEM((F, q_per_w, 128), jnp.int32),
            pltpu.VMEM((128, D), jnp.float32),
            pltpu.VMEM((128,), jnp.float32),
            pltpu.VMEM((rows_w,), jnp.float32),
            pltpu.SemaphoreType.DMA,
        ],
    )


def _tc_body(F, D, flat_ref, fo_ref, num_ref, wnum_ref, bnum_ref,
             w0e_ref, w0n_ref, b0_ref, w1_ref, b1_ref, w2_ref, b2_ref,
             wh_ref, wfm_ref, bout_ref, out_ref):
    x = flat_ref[...]

    x0 = x[:, 0:D]
    s = x0
    sq = x0 * x0
    for f in range(1, F):
        xf = x[:, f * D:(f + 1) * D]
        s = s + xf
        sq = sq + xf * xf
    fm2 = 0.5 * jnp.sum(s * s - sq, axis=1, keepdims=True)

    numeric = num_ref[...]
    fm1 = jnp.dot(numeric, wnum_ref[...]) + bnum_ref[...] + fo_ref[...]
    fm = fm1 + fm2

    h = jnp.dot(x, w0e_ref[...]) + jnp.dot(numeric, w0n_ref[...]) + b0_ref[...]
    h = jnp.maximum(h, 0.0)
    h = jnp.maximum(jnp.dot(h, w1_ref[...]) + b1_ref[...], 0.0)
    h = jnp.maximum(jnp.dot(h, w2_ref[...]) + b2_ref[...], 0.0)

    total = fm * wfm_ref[...] + jnp.dot(h, wh_ref[...]) + bout_ref[...]
    out_ref[...] = 1.0 / (1.0 + jnp.exp(-total))


def kernel(numeric, categorical, W_num, b_num, fo_tables, so_tables,
           W0, b0, W1, b1, W2, b2, Wout, bout):
    B, ND = numeric.shape
    _, F = categorical.shape
    _, V, D = so_tables.shape
    H1 = W0.shape[1]
    H2 = W1.shape[1]
    H3 = W2.shape[1]

    cidx = categorical.astype(jnp.int32) + (jnp.arange(F, dtype=jnp.int32) * V)[None, :]
    idx_w = (B // _NW) * F
    idx_rm = cidx.reshape(_NW, idx_w // 128, 128)
    idx_fm = (cidx.T.reshape(F, _NW, (B // _NW) // 128, 128)
              .transpose(1, 0, 2, 3))
    fo_flat = fo_tables.reshape(F * V)
    fo_pad_len = ((F * V + 7) // 8) * 8
    fo_flat = jnp.concatenate(
        [fo_flat, jnp.zeros((fo_pad_len - F * V,), jnp.float32)])
    so_flat = so_tables.reshape(F * V, D)

    flat_bf, fo_sum = _sc_gather_fn(B, F, V, D, fo_pad_len)(
        idx_rm, idx_fm, fo_flat, so_flat)
    flat = flat_bf.reshape(B, F * D)
    fo2 = fo_sum.reshape(B, 1)

    w0n = W0[:ND]
    w0e = W0[ND:]
    wh = Wout[1:]
    wfm = Wout[0:1]
    bnum = b_num.reshape(1, 1)
    boutr = bout.reshape(1, 1)
    b0r = b0.reshape(1, H1)
    b1r = b1.reshape(1, H2)
    b2r = b2.reshape(1, H3)

    BR = 256
    grid = (B // BR,)

    def full(shape):
        return pl.BlockSpec(shape, lambda i: (0,) * len(shape))

    out2 = pl.pallas_call(
        functools.partial(_tc_body, F, D),
        grid=grid,
        in_specs=[
            pl.BlockSpec((BR, F * D), lambda i: (i, 0)),
            pl.BlockSpec((BR, 1), lambda i: (i, 0)),
            pl.BlockSpec((BR, ND), lambda i: (i, 0)),
            full((ND, 1)),
            full((1, 1)),
            full((F * D, H1)),
            full((ND, H1)),
            full((1, H1)),
            full((H1, H2)),
            full((1, H2)),
            full((H2, H3)),
            full((1, H3)),
            full((H3, 1)),
            full((1, 1)),
            full((1, 1)),
        ],
        out_specs=pl.BlockSpec((BR, 1), lambda i: (i, 0)),
        out_shape=jax.ShapeDtypeStruct((B, 1), jnp.float32),
    )(flat, fo2, numeric, W_num, bnum, w0e, w0n, b0r, W1, b1r, W2, b2r,
      wh, wfm, boutr)

    return out2.reshape(B)


# speedup vs baseline: 27.6745x; 1.5895x over previous
import functools

import jax
import jax.numpy as jnp
from jax import lax
from jax.experimental import pallas as pl
from jax.experimental.pallas import tpu as pltpu
from jax.experimental.pallas import tpu_sc as plsc

_INFO = plsc.get_sparse_core_info()
_NC = _INFO.num_cores
_NS = _INFO.num_subcores
_NW = _NC * _NS


def _sc_gather_fn(B, F, V, D):
    rows_w = B // _NW
    q_per_w = rows_w // 128
    n_chunks = F * q_per_w
    n_fo = rows_w // 16

    mesh = plsc.VectorSubcoreMesh(core_axis_name="c", subcore_axis_name="s")

    def body(idx_hbm, fo_hbm, so_hbm, flat_out, fo_out,
             idx_v, rows_v, val_v, acc_v, sem_so, sem_fo):
        wid = lax.axis_index("s") * _NC + lax.axis_index("c")
        pltpu.sync_copy(idx_hbm.at[wid], idx_v)
        base = wid * rows_w

        def fire(t, b):
            f = t // q_per_w
            q = lax.rem(t, q_per_w)
            pltpu.async_copy(so_hbm.at[idx_v.at[f, q]], rows_v.at[b],
                             sem_so.at[b])
            pltpu.async_copy(fo_hbm.at[idx_v.at[f, q]], val_v.at[b],
                             sem_fo.at[b])

        def zinit(j, carry):
            acc_v[pl.ds(j * 16, 16)] = jnp.zeros((16,), jnp.float32)
            return carry
        lax.fori_loop(0, n_fo, zinit, 0)

        fire(0, 0)

        def chunk(t, carry):
            b = lax.rem(t, 2)
            f = t // q_per_w
            q = lax.rem(t, q_per_w)

            @pl.when(t + 1 < n_chunks)
            def _():
                fire(t + 1, lax.rem(t + 1, 2))

            pltpu.make_async_copy(so_hbm.at[idx_v.at[f, q]], rows_v.at[b],
                                  sem_so.at[b]).wait()
            pltpu.sync_copy(rows_v.at[b],
                            flat_out.at[f, pl.ds(base + q * 128, 128)])
            pltpu.make_async_copy(fo_hbm.at[idx_v.at[f, q]], val_v.at[b],
                                  sem_fo.at[b]).wait()
            for k in range(8):
                sl = pl.ds(q * 128 + k * 16, 16)
                acc_v[sl] = acc_v[sl] + val_v[b, pl.ds(k * 16, 16)]
            return carry
        lax.fori_loop(0, n_chunks, chunk, 0)

        pltpu.sync_copy(acc_v, fo_out.at[pl.ds(base, rows_w)])

    return pl.kernel(
        body,
        mesh=mesh,
        out_type=(
            jax.ShapeDtypeStruct((F, B, D), jnp.float32),
            jax.ShapeDtypeStruct((B,), jnp.float32),
        ),
        scratch_types=[
            pltpu.VMEM((F, q_per_w, 128), jnp.int32),
            pltpu.VMEM((2, 128, D), jnp.float32),
            pltpu.VMEM((2, 128), jnp.float32),
            pltpu.VMEM((rows_w,), jnp.float32),
            pltpu.SemaphoreType.DMA((2,)),
            pltpu.SemaphoreType.DMA((2,)),
        ],
    )


def _tc_body(F, D, flat_ref, fo_ref, num_ref, wnum_ref, bnum_ref,
             w0e_ref, w0n_ref, b0_ref, w1_ref, b1_ref, w2_ref, b2_ref,
             wh_ref, wfm_ref, bout_ref, out_ref):
    x3 = flat_ref[...]
    xs = [x3[f] for f in range(F)]
    x2 = jnp.concatenate(xs, axis=1)

    s = xs[0]
    sq = xs[0] * xs[0]
    for f in range(1, F):
        s = s + xs[f]
        sq = sq + xs[f] * xs[f]
    fm2 = 0.5 * jnp.sum(s * s - sq, axis=1, keepdims=True)

    numeric = num_ref[...]
    fm1 = jnp.dot(numeric, wnum_ref[...]) + bnum_ref[...] + fo_ref[...]
    fm = fm1 + fm2

    h = jnp.dot(x2, w0e_ref[...]) + jnp.dot(numeric, w0n_ref[...]) + b0_ref[...]
    h = jnp.maximum(h, 0.0)
    h = jnp.maximum(jnp.dot(h, w1_ref[...]) + b1_ref[...], 0.0)
    h = jnp.maximum(jnp.dot(h, w2_ref[...]) + b2_ref[...], 0.0)

    total = fm * wfm_ref[...] + jnp.dot(h, wh_ref[...]) + bout_ref[...]
    out_ref[...] = 1.0 / (1.0 + jnp.exp(-total))


def kernel(numeric, categorical, W_num, b_num, fo_tables, so_tables,
           W0, b0, W1, b1, W2, b2, Wout, bout):
    B, ND = numeric.shape
    _, F = categorical.shape
    _, V, D = so_tables.shape
    H1 = W0.shape[1]
    H2 = W1.shape[1]
    H3 = W2.shape[1]

    cidx = categorical.astype(jnp.int32) + (jnp.arange(F, dtype=jnp.int32) * V)[None, :]
    rows_w = B // _NW
    idx_fm = (cidx.T.reshape(F, _NW, rows_w // 128, 128)
              .transpose(1, 0, 2, 3))
    fo_flat = fo_tables.reshape(F * V)
    fo_pad_len = ((F * V + 7) // 8) * 8
    fo_flat = jnp.concatenate(
        [fo_flat, jnp.zeros((fo_pad_len - F * V,), jnp.float32)])
    so_flat = so_tables.reshape(F * V, D)

    flat3, fo_sum = _sc_gather_fn(B, F, V, D)(idx_fm, fo_flat, so_flat)
    fo2 = fo_sum.reshape(B, 1)

    w0n = W0[:ND]
    w0e = W0[ND:]
    wh = Wout[1:]
    wfm = Wout[0:1]
    bnum = b_num.reshape(1, 1)
    boutr = bout.reshape(1, 1)
    b0r = b0.reshape(1, H1)
    b1r = b1.reshape(1, H2)
    b2r = b2.reshape(1, H3)

    BR = 256
    grid = (B // BR,)

    def full(shape):
        return pl.BlockSpec(shape, lambda i: (0,) * len(shape))

    out2 = pl.pallas_call(
        functools.partial(_tc_body, F, D),
        grid=grid,
        in_specs=[
            pl.BlockSpec((F, BR, D), lambda i: (0, i, 0)),
            pl.BlockSpec((BR, 1), lambda i: (i, 0)),
            pl.BlockSpec((BR, ND), lambda i: (i, 0)),
            full((ND, 1)),
            full((1, 1)),
            full((F * D, H1)),
            full((ND, H1)),
            full((1, H1)),
            full((H1, H2)),
            full((1, H2)),
            full((H2, H3)),
            full((1, H3)),
            full((H3, 1)),
            full((1, 1)),
            full((1, 1)),
        ],
        out_specs=pl.BlockSpec((BR, 1), lambda i: (i, 0)),
        out_shape=jax.ShapeDtypeStruct((B, 1), jnp.float32),
    )(flat3, fo2, numeric, W_num, bnum, w0e, w0n, b0r, W1, b1r, W2, b2r,
      wh, wfm, boutr)

    return out2.reshape(B)


# speedup vs baseline: 28.6170x; 1.0341x over previous
import functools

import jax
import jax.numpy as jnp
from jax import lax
from jax.experimental import pallas as pl
from jax.experimental.pallas import tpu as pltpu
from jax.experimental.pallas import tpu_sc as plsc

_INFO = plsc.get_sparse_core_info()
_NC = _INFO.num_cores
_NS = _INFO.num_subcores
_NW = _NC * _NS


def _sc_gather_fn(B, F, V, D):
    rows_w = B // _NW
    q_per_w = rows_w // 128
    n_chunks = F * q_per_w
    n_fo = rows_w // 16

    mesh = plsc.VectorSubcoreMesh(core_axis_name="c", subcore_axis_name="s")

    h_per_w = rows_w // 256
    n_super = F * h_per_w

    def body(idx_hbm, fo_hbm, so_hbm, flat_out, fo_out,
             idx_v, rows_v, val_v, acc_v, sem_so, sem_fo):
        wid = lax.axis_index("s") * _NC + lax.axis_index("c")
        pltpu.sync_copy(idx_hbm.at[wid], idx_v)
        base = wid * rows_w

        def fire(u, b):
            f = u // h_per_w
            h = lax.rem(u, h_per_w)
            for j in range(2):
                q = h * 2 + j
                pltpu.async_copy(so_hbm.at[idx_v.at[f, q]],
                                 rows_v.at[b, pl.ds(j * 128, 128)],
                                 sem_so.at[b])
                pltpu.async_copy(fo_hbm.at[idx_v.at[f, q]],
                                 val_v.at[b, j], sem_fo.at[b])

        def zinit(j, carry):
            acc_v[pl.ds(j * 16, 16)] = jnp.zeros((16,), jnp.float32)
            return carry
        lax.fori_loop(0, n_fo, zinit, 0)

        fire(0, 0)

        def chunk(u, carry):
            b = lax.rem(u, 2)
            f = u // h_per_w
            h = lax.rem(u, h_per_w)

            @pl.when(u + 1 < n_super)
            def _():
                fire(u + 1, lax.rem(u + 1, 2))

            for j in range(2):
                pltpu.make_async_copy(so_hbm.at[pl.ds(0, 128)],
                                      rows_v.at[b, pl.ds(j * 128, 128)],
                                      sem_so.at[b]).wait()
            pltpu.sync_copy(rows_v.at[b],
                            flat_out.at[f, pl.ds(base + h * 256, 256)])
            for j in range(2):
                pltpu.make_async_copy(fo_hbm.at[pl.ds(0, 128)],
                                      val_v.at[b, j], sem_fo.at[b]).wait()
            for j in range(2):
                for k in range(8):
                    sl = pl.ds(h * 256 + j * 128 + k * 16, 16)
                    acc_v[sl] = acc_v[sl] + val_v[b, j, pl.ds(k * 16, 16)]
            return carry
        lax.fori_loop(0, n_super, chunk, 0)

        pltpu.sync_copy(acc_v, fo_out.at[pl.ds(base, rows_w)])

    return pl.kernel(
        body,
        mesh=mesh,
        out_type=(
            jax.ShapeDtypeStruct((F, B, D), jnp.float32),
            jax.ShapeDtypeStruct((B,), jnp.float32),
        ),
        scratch_types=[
            pltpu.VMEM((F, q_per_w, 128), jnp.int32),
            pltpu.VMEM((2, 256, D), jnp.float32),
            pltpu.VMEM((2, 2, 128), jnp.float32),
            pltpu.VMEM((rows_w,), jnp.float32),
            pltpu.SemaphoreType.DMA((2,)),
            pltpu.SemaphoreType.DMA((2,)),
        ],
    )


def _tc_body(F, D, flat_ref, fo_ref, num_ref, wnum_ref, bnum_ref,
             w0e_ref, w0n_ref, b0_ref, w1_ref, b1_ref, w2_ref, b2_ref,
             wh_ref, wfm_ref, bout_ref, out_ref):
    x3 = flat_ref[...]
    xs = [x3[f] for f in range(F)]
    x2 = jnp.concatenate(xs, axis=1)

    s = xs[0]
    sq = xs[0] * xs[0]
    for f in range(1, F):
        s = s + xs[f]
        sq = sq + xs[f] * xs[f]
    fm2 = 0.5 * jnp.sum(s * s - sq, axis=1, keepdims=True)

    numeric = num_ref[...]
    fm1 = jnp.dot(numeric, wnum_ref[...]) + bnum_ref[...] + fo_ref[...]
    fm = fm1 + fm2

    def bdot(a, w):
        return jax.lax.dot_general(
            a.astype(jnp.bfloat16), w,
            (((1,), (0,)), ((), ())),
            preferred_element_type=jnp.float32)

    h = bdot(x2, w0e_ref[...]) + jnp.dot(numeric, w0n_ref[...]) + b0_ref[...]
    h = jnp.maximum(h, 0.0)
    h = jnp.maximum(bdot(h, w1_ref[...]) + b1_ref[...], 0.0)
    h = jnp.maximum(bdot(h, w2_ref[...]) + b2_ref[...], 0.0)

    total = fm * wfm_ref[...] + jnp.dot(h, wh_ref[...]) + bout_ref[...]
    out_ref[...] = 1.0 / (1.0 + jnp.exp(-total))


def kernel(numeric, categorical, W_num, b_num, fo_tables, so_tables,
           W0, b0, W1, b1, W2, b2, Wout, bout):
    B, ND = numeric.shape
    _, F = categorical.shape
    _, V, D = so_tables.shape
    H1 = W0.shape[1]
    H2 = W1.shape[1]
    H3 = W2.shape[1]

    cidx = categorical.astype(jnp.int32) + (jnp.arange(F, dtype=jnp.int32) * V)[None, :]
    rows_w = B // _NW
    idx_fm = (cidx.T.reshape(F, _NW, rows_w // 128, 128)
              .transpose(1, 0, 2, 3))
    fo_flat = fo_tables.reshape(F * V)
    fo_pad_len = ((F * V + 7) // 8) * 8
    fo_flat = jnp.concatenate(
        [fo_flat, jnp.zeros((fo_pad_len - F * V,), jnp.float32)])
    so_flat = so_tables.reshape(F * V, D)

    flat3, fo_sum = _sc_gather_fn(B, F, V, D)(idx_fm, fo_flat, so_flat)
    fo2 = fo_sum.reshape(B, 1)

    w0n = W0[:ND]
    w0e = W0[ND:].astype(jnp.bfloat16)
    W1b = W1.astype(jnp.bfloat16)
    W2b = W2.astype(jnp.bfloat16)
    wh = Wout[1:]
    wfm = Wout[0:1]
    bnum = b_num.reshape(1, 1)
    boutr = bout.reshape(1, 1)
    b0r = b0.reshape(1, H1)
    b1r = b1.reshape(1, H2)
    b2r = b2.reshape(1, H3)

    BR = 256
    grid = (B // BR,)

    def full(shape):
        return pl.BlockSpec(shape, lambda i: (0,) * len(shape))

    out2 = pl.pallas_call(
        functools.partial(_tc_body, F, D),
        grid=grid,
        in_specs=[
            pl.BlockSpec((F, BR, D), lambda i: (0, i, 0)),
            pl.BlockSpec((BR, 1), lambda i: (i, 0)),
            pl.BlockSpec((BR, ND), lambda i: (i, 0)),
            full((ND, 1)),
            full((1, 1)),
            full((F * D, H1)),
            full((ND, H1)),
            full((1, H1)),
            full((H1, H2)),
            full((1, H2)),
            full((H2, H3)),
            full((1, H3)),
            full((H3, 1)),
            full((1, 1)),
            full((1, 1)),
        ],
        out_specs=pl.BlockSpec((BR, 1), lambda i: (i, 0)),
        out_shape=jax.ShapeDtypeStruct((B, 1), jnp.float32),
    )(flat3, fo2, numeric, W_num, bnum, w0e, w0n, b0r, W1b, b1r, W2b, b2r,
      wh, wfm, boutr)

    return out2.reshape(B)


# speedup vs baseline: 28.9700x; 1.0123x over previous
import functools

import jax
import jax.numpy as jnp
from jax import lax
from jax.experimental import pallas as pl
from jax.experimental.pallas import tpu as pltpu
from jax.experimental.pallas import tpu_sc as plsc

_INFO = plsc.get_sparse_core_info()
_NC = _INFO.num_cores
_NS = _INFO.num_subcores
_NW = _NC * _NS


def _sc_gather_fn(B, F, V, D):
    rows_w = B // _NW
    q_per_w = rows_w // 128
    n_chunks = F * q_per_w
    n_fo = rows_w // 16

    mesh = plsc.VectorSubcoreMesh(core_axis_name="c", subcore_axis_name="s")

    h_per_w = rows_w // 256
    n_super = F * h_per_w

    NB = 3

    def body(idx_hbm, fo_hbm, so_hbm, flat_out, fo_out,
             idx_v, rows_v, val_v, acc_v, sem_so, sem_fo, sem_wr):
        wid = lax.axis_index("s") * _NC + lax.axis_index("c")
        pltpu.sync_copy(idx_hbm.at[wid], idx_v)
        base = wid * rows_w

        def fire(u, b):
            f = u // h_per_w
            h = lax.rem(u, h_per_w)
            for j in range(2):
                q = h * 2 + j
                pltpu.async_copy(so_hbm.at[idx_v.at[f, q]],
                                 rows_v.at[b, pl.ds(j * 128, 128)],
                                 sem_so.at[b])
                pltpu.async_copy(fo_hbm.at[idx_v.at[f, q]],
                                 val_v.at[b, j], sem_fo.at[b])

        def wait_write(b):
            pltpu.make_async_copy(rows_v.at[b],
                                  flat_out.at[0, pl.ds(0, 256)],
                                  sem_wr.at[b]).wait()

        def zinit(j, carry):
            acc_v[pl.ds(j * 16, 16)] = jnp.zeros((16,), jnp.float32)
            return carry
        lax.fori_loop(0, n_fo, zinit, 0)

        for u0 in range(2):
            fire(u0, u0)

        def chunk(u, carry):
            b = lax.rem(u, NB)
            f = u // h_per_w
            h = lax.rem(u, h_per_w)

            for j in range(2):
                pltpu.make_async_copy(so_hbm.at[pl.ds(0, 128)],
                                      rows_v.at[b, pl.ds(j * 128, 128)],
                                      sem_so.at[b]).wait()
            pltpu.async_copy(rows_v.at[b],
                             flat_out.at[f, pl.ds(base + h * 256, 256)],
                             sem_wr.at[b])

            @pl.when(u + 2 < n_super)
            def _():
                b2 = lax.rem(u + 2, NB)

                @pl.when(u >= 1)
                def _():
                    wait_write(b2)
                fire(u + 2, b2)

            for j in range(2):
                pltpu.make_async_copy(fo_hbm.at[pl.ds(0, 128)],
                                      val_v.at[b, j], sem_fo.at[b]).wait()
            for j in range(2):
                for k in range(8):
                    sl = pl.ds(h * 256 + j * 128 + k * 16, 16)
                    acc_v[sl] = acc_v[sl] + val_v[b, j, pl.ds(k * 16, 16)]
            return carry
        lax.fori_loop(0, n_super, chunk, 0)

        for t in range(NB):
            u = n_super - NB + t
            if u >= 0:
                wait_write(u % NB)

        pltpu.sync_copy(acc_v, fo_out.at[pl.ds(base, rows_w)])

    return pl.kernel(
        body,
        mesh=mesh,
        out_type=(
            jax.ShapeDtypeStruct((F, B, D), jnp.float32),
            jax.ShapeDtypeStruct((B,), jnp.float32),
        ),
        scratch_types=[
            pltpu.VMEM((F, q_per_w, 128), jnp.int32),
            pltpu.VMEM((NB, 256, D), jnp.float32),
            pltpu.VMEM((NB, 2, 128), jnp.float32),
            pltpu.VMEM((rows_w,), jnp.float32),
            pltpu.SemaphoreType.DMA((NB,)),
            pltpu.SemaphoreType.DMA((NB,)),
            pltpu.SemaphoreType.DMA((NB,)),
        ],
    )


def _tc_body(F, D, flat_ref, fo_ref, num_ref, wnum_ref, bnum_ref,
             w0e_ref, w0n_ref, b0_ref, w1_ref, b1_ref, w2_ref, b2_ref,
             wh_ref, wfm_ref, bout_ref, out_ref):
    x3 = flat_ref[...]
    xs = [x3[f] for f in range(F)]
    x2 = jnp.concatenate(xs, axis=1)

    s = xs[0]
    sq = xs[0] * xs[0]
    for f in range(1, F):
        s = s + xs[f]
        sq = sq + xs[f] * xs[f]
    fm2 = 0.5 * jnp.sum(s * s - sq, axis=1, keepdims=True)

    numeric = num_ref[...]
    fm1 = jnp.dot(numeric, wnum_ref[...]) + bnum_ref[...] + fo_ref[...]
    fm = fm1 + fm2

    def bdot(a, w):
        return jax.lax.dot_general(
            a.astype(jnp.bfloat16), w,
            (((1,), (0,)), ((), ())),
            preferred_element_type=jnp.float32)

    h = bdot(x2, w0e_ref[...]) + jnp.dot(numeric, w0n_ref[...]) + b0_ref[...]
    h = jnp.maximum(h, 0.0)
    h = jnp.maximum(bdot(h, w1_ref[...]) + b1_ref[...], 0.0)
    h = jnp.maximum(bdot(h, w2_ref[...]) + b2_ref[...], 0.0)

    total = fm * wfm_ref[...] + jnp.dot(h, wh_ref[...]) + bout_ref[...]
    out_ref[...] = 1.0 / (1.0 + jnp.exp(-total))


def kernel(numeric, categorical, W_num, b_num, fo_tables, so_tables,
           W0, b0, W1, b1, W2, b2, Wout, bout):
    B, ND = numeric.shape
    _, F = categorical.shape
    _, V, D = so_tables.shape
    H1 = W0.shape[1]
    H2 = W1.shape[1]
    H3 = W2.shape[1]

    cidx = categorical.astype(jnp.int32) + (jnp.arange(F, dtype=jnp.int32) * V)[None, :]
    rows_w = B // _NW
    idx_fm = (cidx.T.reshape(F, _NW, rows_w // 128, 128)
              .transpose(1, 0, 2, 3))
    fo_flat = fo_tables.reshape(F * V)
    fo_pad_len = ((F * V + 7) // 8) * 8
    fo_flat = jnp.concatenate(
        [fo_flat, jnp.zeros((fo_pad_len - F * V,), jnp.float32)])
    so_flat = so_tables.reshape(F * V, D)

    flat3, fo_sum = _sc_gather_fn(B, F, V, D)(idx_fm, fo_flat, so_flat)
    fo2 = fo_sum.reshape(B, 1)

    w0n = W0[:ND]
    w0e = W0[ND:].astype(jnp.bfloat16)
    W1b = W1.astype(jnp.bfloat16)
    W2b = W2.astype(jnp.bfloat16)
    wh = Wout[1:]
    wfm = Wout[0:1]
    bnum = b_num.reshape(1, 1)
    boutr = bout.reshape(1, 1)
    b0r = b0.reshape(1, H1)
    b1r = b1.reshape(1, H2)
    b2r = b2.reshape(1, H3)

    BR = 256
    grid = (B // BR,)

    def full(shape):
        return pl.BlockSpec(shape, lambda i: (0,) * len(shape))

    out2 = pl.pallas_call(
        functools.partial(_tc_body, F, D),
        grid=grid,
        in_specs=[
            pl.BlockSpec((F, BR, D), lambda i: (0, i, 0)),
            pl.BlockSpec((BR, 1), lambda i: (i, 0)),
            pl.BlockSpec((BR, ND), lambda i: (i, 0)),
            full((ND, 1)),
            full((1, 1)),
            full((F * D, H1)),
            full((ND, H1)),
            full((1, H1)),
            full((H1, H2)),
            full((1, H2)),
            full((H2, H3)),
            full((1, H3)),
            full((H3, 1)),
            full((1, 1)),
            full((1, 1)),
        ],
        out_specs=pl.BlockSpec((BR, 1), lambda i: (i, 0)),
        out_shape=jax.ShapeDtypeStruct((B, 1), jnp.float32),
    )(flat3, fo2, numeric, W_num, bnum, w0e, w0n, b0r, W1b, b1r, W2b, b2r,
      wh, wfm, boutr)

    return out2.reshape(B)


# speedup vs baseline: 33.3459x; 1.1510x over previous
import functools

import jax
import jax.numpy as jnp
from jax import lax
from jax.experimental import pallas as pl
from jax.experimental.pallas import tpu as pltpu
from jax.experimental.pallas import tpu_sc as plsc

_INFO = plsc.get_sparse_core_info()
_NC = _INFO.num_cores
_NS = _INFO.num_subcores
_NW = _NC * _NS


def _sc_gather_fn(B, F, V, D):
    rows_w = B // _NW
    q_per_w = rows_w // 128
    n_chunks = F * q_per_w
    n_fo = rows_w // 16

    mesh = plsc.VectorSubcoreMesh(core_axis_name="c", subcore_axis_name="s")

    h_per_w = rows_w // 256
    n_super = F * h_per_w

    NB = 3

    def body(idx_hbm, fo_hbm, so_hbm, flat_out, fo_out,
             idx_v, rows_v, val_v, acc_v, sem_so, sem_fo, sem_wr):
        wid = lax.axis_index("s") * _NC + lax.axis_index("c")
        pltpu.sync_copy(idx_hbm.at[wid], idx_v)
        base = wid * rows_w

        def fire(u, b):
            f = u // h_per_w
            h = lax.rem(u, h_per_w)
            for j in range(2):
                q = h * 2 + j
                pltpu.async_copy(so_hbm.at[idx_v.at[f, q]],
                                 rows_v.at[b, pl.ds(j * 128, 128)],
                                 sem_so.at[b])
                pltpu.async_copy(fo_hbm.at[idx_v.at[f, q]],
                                 val_v.at[b, j], sem_fo.at[b])

        def wait_write(b):
            pltpu.make_async_copy(rows_v.at[b],
                                  flat_out.at[0, pl.ds(0, 256)],
                                  sem_wr.at[b]).wait()

        def zinit(j, carry):
            acc_v[pl.ds(j * 16, 16)] = jnp.zeros((16,), jnp.float32)
            return carry
        lax.fori_loop(0, n_fo, zinit, 0)

        for u0 in range(2):
            fire(u0, u0)

        def chunk(u, carry):
            b = lax.rem(u, NB)
            f = u // h_per_w
            h = lax.rem(u, h_per_w)

            for j in range(2):
                pltpu.make_async_copy(so_hbm.at[pl.ds(0, 128)],
                                      rows_v.at[b, pl.ds(j * 128, 128)],
                                      sem_so.at[b]).wait()
            pltpu.async_copy(rows_v.at[b],
                             flat_out.at[f, pl.ds(base + h * 256, 256)],
                             sem_wr.at[b])

            @pl.when(u + 2 < n_super)
            def _():
                b2 = lax.rem(u + 2, NB)

                @pl.when(u >= 1)
                def _():
                    wait_write(b2)
                fire(u + 2, b2)

            for j in range(2):
                pltpu.make_async_copy(fo_hbm.at[pl.ds(0, 128)],
                                      val_v.at[b, j], sem_fo.at[b]).wait()
            for j in range(2):
                for k in range(8):
                    sl = pl.ds(h * 256 + j * 128 + k * 16, 16)
                    acc_v[sl] = acc_v[sl] + val_v[b, j, pl.ds(k * 16, 16)]
            return carry
        lax.fori_loop(0, n_super, chunk, 0)

        for t in range(NB):
            u = n_super - NB + t
            if u >= 0:
                wait_write(u % NB)

        pltpu.sync_copy(acc_v, fo_out.at[pl.ds(base, rows_w)])

    return pl.kernel(
        body,
        mesh=mesh,
        out_type=(
            jax.ShapeDtypeStruct((F, B, D), jnp.float32),
            jax.ShapeDtypeStruct((B,), jnp.float32),
        ),
        scratch_types=[
            pltpu.VMEM((F, q_per_w, 128), jnp.int32),
            pltpu.VMEM((NB, 256, D), jnp.float32),
            pltpu.VMEM((NB, 2, 128), jnp.float32),
            pltpu.VMEM((rows_w,), jnp.float32),
            pltpu.SemaphoreType.DMA((NB,)),
            pltpu.SemaphoreType.DMA((NB,)),
            pltpu.SemaphoreType.DMA((NB,)),
        ],
    )


def _tc_body(F, D, flat_ref, fo_ref, num_ref, wnum_ref, bnum_ref,
             w0e_ref, w0n_ref, b0_ref, w1_ref, b1_ref, w2_ref, b2_ref,
             wh_ref, wfm_ref, bout_ref, out_ref):
    x3 = flat_ref[...]
    xs = [x3[f] for f in range(F)]
    x2 = jnp.concatenate(xs, axis=1)

    s = xs[0]
    sq = xs[0] * xs[0]
    for f in range(1, F):
        s = s + xs[f]
        sq = sq + xs[f] * xs[f]
    fm2 = 0.5 * jnp.sum(s * s - sq, axis=1, keepdims=True)

    numeric = num_ref[...]
    fm1 = jnp.dot(numeric, wnum_ref[...]) + bnum_ref[...] + fo_ref[...]
    fm = fm1 + fm2

    def bdot(a, w):
        return jax.lax.dot_general(
            a.astype(jnp.bfloat16), w,
            (((1,), (0,)), ((), ())),
            preferred_element_type=jnp.float32)

    h = bdot(x2, w0e_ref[...]) + jnp.dot(numeric, w0n_ref[...]) + b0_ref[...]
    h = jnp.maximum(h, 0.0)
    h = jnp.maximum(bdot(h, w1_ref[...]) + b1_ref[...], 0.0)
    h = jnp.maximum(bdot(h, w2_ref[...]) + b2_ref[...], 0.0)

    total = fm * wfm_ref[...] + jnp.dot(h, wh_ref[...]) + bout_ref[...]
    out_ref[...] = 1.0 / (1.0 + jnp.exp(-total))


def kernel(numeric, categorical, W_num, b_num, fo_tables, so_tables,
           W0, b0, W1, b1, W2, b2, Wout, bout):
    B, ND = numeric.shape
    _, F = categorical.shape
    _, V, D = so_tables.shape
    H1 = W0.shape[1]
    H2 = W1.shape[1]
    H3 = W2.shape[1]

    NSPLIT = 2
    Bh = B // NSPLIT

    cidx = categorical.astype(jnp.int32) + (jnp.arange(F, dtype=jnp.int32) * V)[None, :]
    fo_flat = fo_tables.reshape(F * V)
    fo_pad_len = ((F * V + 7) // 8) * 8
    fo_flat = jnp.concatenate(
        [fo_flat, jnp.zeros((fo_pad_len - F * V,), jnp.float32)])
    so_flat = so_tables.reshape(F * V, D)

    rows_w = Bh // _NW
    sc_fn = _sc_gather_fn(Bh, F, V, D)
    halves = []
    for p in range(NSPLIT):
        cidx_h = cidx[p * Bh:(p + 1) * Bh]
        idx_fm = (cidx_h.T.reshape(F, _NW, rows_w // 128, 128)
                  .transpose(1, 0, 2, 3))
        halves.append(sc_fn(idx_fm, fo_flat, so_flat))

    w0n = W0[:ND]
    w0e = W0[ND:].astype(jnp.bfloat16)
    W1b = W1.astype(jnp.bfloat16)
    W2b = W2.astype(jnp.bfloat16)
    wh = Wout[1:]
    wfm = Wout[0:1]
    bnum = b_num.reshape(1, 1)
    boutr = bout.reshape(1, 1)
    b0r = b0.reshape(1, H1)
    b1r = b1.reshape(1, H2)
    b2r = b2.reshape(1, H3)

    BR = 256
    grid = (Bh // BR,)

    def full(shape):
        return pl.BlockSpec(shape, lambda i: (0,) * len(shape))

    tc_call = pl.pallas_call(
        functools.partial(_tc_body, F, D),
        grid=grid,
        in_specs=[
            pl.BlockSpec((F, BR, D), lambda i: (0, i, 0)),
            pl.BlockSpec((BR, 1), lambda i: (i, 0)),
            pl.BlockSpec((BR, ND), lambda i: (i, 0)),
            full((ND, 1)),
            full((1, 1)),
            full((F * D, H1)),
            full((ND, H1)),
            full((1, H1)),
            full((H1, H2)),
            full((1, H2)),
            full((H2, H3)),
            full((1, H3)),
            full((H3, 1)),
            full((1, 1)),
            full((1, 1)),
        ],
        out_specs=pl.BlockSpec((BR, 1), lambda i: (i, 0)),
        out_shape=jax.ShapeDtypeStruct((Bh, 1), jnp.float32),
    )

    outs = []
    for p in range(NSPLIT):
        flat3, fo_sum = halves[p]
        fo2 = fo_sum.reshape(Bh, 1)
        numeric_h = numeric[p * Bh:(p + 1) * Bh]
        outs.append(tc_call(
            flat3, fo2, numeric_h, W_num, bnum, w0e, w0n, b0r, W1b, b1r,
            W2b, b2r, wh, wfm, boutr))

    return jnp.concatenate(outs, axis=0).reshape(B)


# speedup vs baseline: 34.0000x; 1.0196x over previous
import functools

import jax
import jax.numpy as jnp
from jax import lax
from jax.experimental import pallas as pl
from jax.experimental.pallas import tpu as pltpu
from jax.experimental.pallas import tpu_sc as plsc

_INFO = plsc.get_sparse_core_info()
_NC = _INFO.num_cores
_NS = _INFO.num_subcores
_NW = _NC * _NS


def _sc_gather_fn(B, F, V, D):
    rows_w = B // _NW
    q_per_w = rows_w // 128
    n_chunks = F * q_per_w
    n_fo = rows_w // 16

    mesh = plsc.VectorSubcoreMesh(core_axis_name="c", subcore_axis_name="s")

    SUB = 2 if q_per_w % 2 == 0 else 1
    h_per_w = q_per_w // SUB
    n_super = F * h_per_w
    CR = SUB * 128

    NB = 3

    def body(idx_hbm, fo_hbm, so_hbm, flat_out, fo_out,
             idx_v, rows_v, val_v, acc_v, sem_so, sem_fo, sem_wr):
        wid = lax.axis_index("s") * _NC + lax.axis_index("c")
        pltpu.sync_copy(idx_hbm.at[wid], idx_v)
        base = wid * rows_w

        def fire(u, b):
            f = u // h_per_w
            h = lax.rem(u, h_per_w)
            for j in range(SUB):
                q = h * SUB + j
                pltpu.async_copy(so_hbm.at[idx_v.at[f, q]],
                                 rows_v.at[b, pl.ds(j * 128, 128)],
                                 sem_so.at[b])
                pltpu.async_copy(fo_hbm.at[idx_v.at[f, q]],
                                 val_v.at[b, j], sem_fo.at[b])

        def wait_write(b):
            pltpu.make_async_copy(rows_v.at[b],
                                  flat_out.at[0, pl.ds(0, CR)],
                                  sem_wr.at[b]).wait()

        def zinit(j, carry):
            acc_v[pl.ds(j * 16, 16)] = jnp.zeros((16,), jnp.float32)
            return carry
        lax.fori_loop(0, n_fo, zinit, 0)

        for u0 in range(2):
            fire(u0, u0)

        def chunk(u, carry):
            b = lax.rem(u, NB)
            f = u // h_per_w
            h = lax.rem(u, h_per_w)

            for j in range(SUB):
                pltpu.make_async_copy(so_hbm.at[pl.ds(0, 128)],
                                      rows_v.at[b, pl.ds(j * 128, 128)],
                                      sem_so.at[b]).wait()
            pltpu.async_copy(rows_v.at[b],
                             flat_out.at[f, pl.ds(base + h * CR, CR)],
                             sem_wr.at[b])

            @pl.when(u + 2 < n_super)
            def _():
                b2 = lax.rem(u + 2, NB)

                @pl.when(u >= 1)
                def _():
                    wait_write(b2)
                fire(u + 2, b2)

            for j in range(SUB):
                pltpu.make_async_copy(fo_hbm.at[pl.ds(0, 128)],
                                      val_v.at[b, j], sem_fo.at[b]).wait()
            for j in range(SUB):
                for k in range(8):
                    sl = pl.ds(h * CR + j * 128 + k * 16, 16)
                    acc_v[sl] = acc_v[sl] + val_v[b, j, pl.ds(k * 16, 16)]
            return carry
        lax.fori_loop(0, n_super, chunk, 0)

        for t in range(NB):
            u = n_super - NB + t
            if u >= 0:
                wait_write(u % NB)

        pltpu.sync_copy(acc_v, fo_out.at[pl.ds(base, rows_w)])

    return pl.kernel(
        body,
        mesh=mesh,
        out_type=(
            jax.ShapeDtypeStruct((F, B, D), jnp.float32),
            jax.ShapeDtypeStruct((B,), jnp.float32),
        ),
        scratch_types=[
            pltpu.VMEM((F, q_per_w, 128), jnp.int32),
            pltpu.VMEM((NB, CR, D), jnp.float32),
            pltpu.VMEM((NB, SUB, 128), jnp.float32),
            pltpu.VMEM((rows_w,), jnp.float32),
            pltpu.SemaphoreType.DMA((NB,)),
            pltpu.SemaphoreType.DMA((NB,)),
            pltpu.SemaphoreType.DMA((NB,)),
        ],
    )


def _tc_body(F, D, flat_ref, fo_ref, num_ref, wnum_ref, bnum_ref,
             w0e_ref, w0n_ref, b0_ref, w1_ref, b1_ref, w2_ref, b2_ref,
             wh_ref, wfm_ref, bout_ref, out_ref):
    x3 = flat_ref[...]
    xs = [x3[f] for f in range(F)]
    x2 = jnp.concatenate(xs, axis=1)

    s = xs[0]
    sq = xs[0] * xs[0]
    for f in range(1, F):
        s = s + xs[f]
        sq = sq + xs[f] * xs[f]
    fm2 = 0.5 * jnp.sum(s * s - sq, axis=1, keepdims=True)

    numeric = num_ref[...]
    fm1 = jnp.dot(numeric, wnum_ref[...]) + bnum_ref[...] + fo_ref[...]
    fm = fm1 + fm2

    def bdot(a, w):
        return jax.lax.dot_general(
            a.astype(jnp.bfloat16), w,
            (((1,), (0,)), ((), ())),
            preferred_element_type=jnp.float32)

    h = bdot(x2, w0e_ref[...]) + jnp.dot(numeric, w0n_ref[...]) + b0_ref[...]
    h = jnp.maximum(h, 0.0)
    h = jnp.maximum(bdot(h, w1_ref[...]) + b1_ref[...], 0.0)
    h = jnp.maximum(bdot(h, w2_ref[...]) + b2_ref[...], 0.0)

    total = fm * wfm_ref[...] + jnp.dot(h, wh_ref[...]) + bout_ref[...]
    out_ref[...] = 1.0 / (1.0 + jnp.exp(-total))


def kernel(numeric, categorical, W_num, b_num, fo_tables, so_tables,
           W0, b0, W1, b1, W2, b2, Wout, bout):
    B, ND = numeric.shape
    _, F = categorical.shape
    _, V, D = so_tables.shape
    H1 = W0.shape[1]
    H2 = W1.shape[1]
    H3 = W2.shape[1]

    NSPLIT = 4
    Bh = B // NSPLIT

    cidx = categorical.astype(jnp.int32) + (jnp.arange(F, dtype=jnp.int32) * V)[None, :]
    fo_flat = fo_tables.reshape(F * V)
    fo_pad_len = ((F * V + 7) // 8) * 8
    fo_flat = jnp.concatenate(
        [fo_flat, jnp.zeros((fo_pad_len - F * V,), jnp.float32)])
    so_flat = so_tables.reshape(F * V, D)

    rows_w = Bh // _NW
    idx_all = (cidx.T.reshape(F, NSPLIT, _NW, rows_w // 128, 128)
               .transpose(1, 2, 0, 3, 4))
    sc_fn = _sc_gather_fn(Bh, F, V, D)
    halves = []
    for p in range(NSPLIT):
        halves.append(sc_fn(idx_all[p], fo_flat, so_flat))

    w0n = W0[:ND]
    w0e = W0[ND:].astype(jnp.bfloat16)
    W1b = W1.astype(jnp.bfloat16)
    W2b = W2.astype(jnp.bfloat16)
    wh = Wout[1:]
    wfm = Wout[0:1]
    bnum = b_num.reshape(1, 1)
    boutr = bout.reshape(1, 1)
    b0r = b0.reshape(1, H1)
    b1r = b1.reshape(1, H2)
    b2r = b2.reshape(1, H3)

    BR = 256
    grid = (Bh // BR,)

    def full(shape):
        return pl.BlockSpec(shape, lambda i: (0,) * len(shape))

    tc_call = pl.pallas_call(
        functools.partial(_tc_body, F, D),
        grid=grid,
        in_specs=[
            pl.BlockSpec((F, BR, D), lambda i: (0, i, 0)),
            pl.BlockSpec((BR, 1), lambda i: (i, 0)),
            pl.BlockSpec((BR, ND), lambda i: (i, 0)),
            full((ND, 1)),
            full((1, 1)),
            full((F * D, H1)),
            full((ND, H1)),
            full((1, H1)),
            full((H1, H2)),
            full((1, H2)),
            full((H2, H3)),
            full((1, H3)),
            full((H3, 1)),
            full((1, 1)),
            full((1, 1)),
        ],
        out_specs=pl.BlockSpec((BR, 1), lambda i: (i, 0)),
        out_shape=jax.ShapeDtypeStruct((Bh, 1), jnp.float32),
    )

    outs = []
    for p in range(NSPLIT):
        flat3, fo_sum = halves[p]
        fo2 = fo_sum.reshape(Bh, 1)
        numeric_h = numeric[p * Bh:(p + 1) * Bh]
        outs.append(tc_call(
            flat3, fo2, numeric_h, W_num, bnum, w0e, w0n, b0r, W1b, b1r,
            W2b, b2r, wh, wfm, boutr))

    return jnp.concatenate(outs, axis=0).reshape(B)
